# baseline (speedup 1.0000x reference)
import functools

import jax
import jax.numpy as jnp
from jax import lax
from jax.experimental import pallas as pl
from jax.experimental.pallas import tpu as pltpu
from jax.experimental.pallas import tpu_sc as plsc

N = 10000
E = 160000
ALPHA = 0.1
EPS = 1e-5

_NC = 2
_NS = 16
_EPT = 10240
_EP = _EPT * _NS
_RG = 8
_NB = _EPT // (80 * _RG)
_NR = _EPT // 80
_CP = 624
_RING = 4


def _mesh():
    return plsc.VectorSubcoreMesh(core_axis_name="c", subcore_axis_name="s")


def _sc_scratch(F):
    return [
        pltpu.VMEM_SHARED((N, F), jnp.float32),
        pltpu.VMEM((2, _RG, 80), jnp.int32),
        pltpu.VMEM((2, _RG, 80), jnp.int32),
        pltpu.VMEM((2, _RG, 80), jnp.float32),
        pltpu.VMEM((_RING * 80, F), jnp.float32),
        pltpu.SemaphoreType.DMA((_RING,)),
        pltpu.SemaphoreType.DMA((_RING,)),
        pltpu.SemaphoreType.DMA,
    ]


def _stage_idx(srcs, dsts, ws, src_v, dst_v, w_v, isem, *, g, sid, b, pb,
               tab_off):
    pltpu.async_copy(srcs.at[g, sid, b], src_v.at[pb], isem).wait()
    pltpu.async_copy(dsts.at[g, sid, b], dst_v.at[pb], isem).wait()
    pltpu.async_copy(ws.at[g, sid, b], w_v.at[pb], isem).wait()

    def adj(i, carry):
        for j in range(80 // 16):
            sl = (pb, i, pl.ds(j * 16, 16))
            src_v[sl] = src_v[sl] + tab_off
        return carry

    lax.fori_loop(0, _RG, adj, 0)


def _spmm_pass(table, srcs, dsts, ws, out, acc, src_v, dst_v, w_v, rows_v,
               gsem, ssem, isem, *, g, sid, tab_off, out_off, fv):
    zvec = jnp.zeros((16,), jnp.float32)

    def zrow(i, carry):
        for j in range(fv):
            rows_v[i, pl.ds(j * 16, 16)] = zvec
        return carry

    lax.fori_loop(0, 80, zrow, 0)
    zsrc = rows_v.at[pl.ds(0, 80)]
    for zb in range(7):
        pltpu.async_copy(zsrc, acc.at[pl.ds(sid * _CP + zb * 80, 80)],
                         isem).wait()
    pltpu.async_copy(rows_v.at[pl.ds(0, 64)],
                     acc.at[pl.ds(sid * _CP + 560, 64)], isem).wait()

    @pl.when(sid == _NS - 1)
    def _():
        pltpu.async_copy(zsrc, acc.at[pl.ds(N - 80, 80)], isem).wait()

    plsc.subcore_barrier()

    def gather_desc(t, s):
        b = t // _RG
        pb = lax.rem(b, 2)
        r = t - b * _RG
        return pltpu.make_async_copy(
            table.at[src_v.at[pb, r]],
            rows_v.at[pl.ds(s * 80, 80)], gsem.at[s])

    def scatter_desc(t, s):
        b = t // _RG
        pb = lax.rem(b, 2)
        r = t - b * _RG
        return pltpu.make_async_copy(
            rows_v.at[pl.ds(s * 80, 80)],
            acc.at[dst_v.at[pb, r]], ssem.at[s])

    _stage_idx(srcs, dsts, ws, src_v, dst_v, w_v, isem, g=g, sid=sid,
               b=0, pb=0, tab_off=tab_off)
    gather_desc(0, 0).start()
    gather_desc(1, 1).start()

    def step(t, carry):
        s = lax.rem(t, _RING)
        gather_desc(t, s).wait()

        b = t // _RG
        r = t - b * _RG

        pb = lax.rem(b, 2)

        for q in range(80 // 16):
            wvec = w_v[pb, r, pl.ds(q * 16, 16)]
            for l in range(16):
                wspl = wvec.at[jnp.full((16,), l, jnp.int32)].get(
                    mode="promise_in_bounds")
                i = q * 16 + l
                for j in range(fv):
                    sl2 = (s * 80 + i, pl.ds(j * 16, 16))
                    rows_v[sl2] = rows_v[sl2] * wspl

        @pl.when(t + 2 < _NR)
        def _():
            s2 = lax.rem(t + 2, _RING)

            @pl.when(t >= 2)
            def _():
                scatter_desc(t - 2, s2).wait()

            @pl.when(lax.rem(t + 2, _RG) == 0)
            def _():
                b2 = (t + 2) // _RG
                _stage_idx(srcs, dsts, ws, src_v, dst_v, w_v, isem, g=g,
                           sid=sid, b=b2, pb=lax.rem(b2, 2),
                           tab_off=tab_off)

            gather_desc(t + 2, s2).start()

        scatter_desc(t, s).start(add=True)
        return carry

    lax.fori_loop(0, _NR, step, 0)
    for dt in range(_RING):
        t = _NR - _RING + dt
        scatter_desc(t, t % _RING).wait()
    plsc.subcore_barrier()
    pltpu.async_copy(acc.at[pl.ds(sid * _CP, _CP)],
                     out.at[pl.ds(out_off + sid * _CP, _CP)], isem).wait()

    @pl.when(sid == _NS - 1)
    def _():
        pltpu.async_copy(acc.at[pl.ds(_NS * _CP, N - _NS * _CP)],
                         out.at[pl.ds(out_off + _NS * _CP, N - _NS * _CP)],
                         isem).wait()

    plsc.subcore_barrier()


def _make_spmm2g(n_ch, F):
    n_pass = n_ch // _NC
    fv = F // 16

    @functools.partial(
        pl.kernel, mesh=_mesh(),
        out_type=jax.ShapeDtypeStruct((2 * n_ch * N, F), jnp.float32),
        scratch_types=_sc_scratch(F),
    )
    def spmm(table, srcs, dsts, ws, out, acc, src_v, dst_v, w_v, rows_v,
             gsem, ssem, isem):
        cid = lax.axis_index("c")
        sid = lax.axis_index("s")

        def one_pass(q, carry):
            g = q // n_pass
            chunk = (q % n_pass) * _NC + cid
            _spmm_pass(table, srcs, dsts, ws, out, acc, src_v, dst_v,
                       w_v, rows_v, gsem, ssem, isem, g=g, sid=sid,
                       tab_off=chunk * N,
                       out_off=(g * n_ch + chunk) * N, fv=fv)
            return carry

        lax.fori_loop(0, 2 * n_pass, one_pass, 0)

    return spmm


def _make_spmm_final(F):
    fv = F // 16

    @functools.partial(
        pl.kernel, mesh=_mesh(),
        out_type=jax.ShapeDtypeStruct((2 * N, F), jnp.float32),
        scratch_types=_sc_scratch(F),
    )
    def spmmf(table, srcs, dsts, ws, out, acc, src_v, dst_v, w_v, rows_v,
              gsem, ssem, isem):
        cid = lax.axis_index("c")
        sid = lax.axis_index("s")
        _spmm_pass(table, srcs, dsts, ws, out, acc, src_v, dst_v, w_v,
                   rows_v, gsem, ssem, isem, g=cid, sid=sid,
                   tab_off=cid * N, out_off=cid * N, fv=fv)

    return spmmf


@functools.lru_cache(maxsize=1)
def _spmm_kernels():
    return _make_spmm2g(2, 128), _make_spmm2g(4, 128), _make_spmm_final(128)


_N_BLK = 400


def _chunk_bn(x, scale, shift):
    def body(x_ref, s_ref, t_ref, o_ref):
        y = x_ref[...] * s_ref[...] + t_ref[...]
        o_ref[0] = y[:, :128]
        o_ref[1] = y[:, 128:]

    return pl.pallas_call(
        body,
        grid=(N // _N_BLK,),
        in_specs=[
            pl.BlockSpec((_N_BLK, 256), lambda n: (n, 0)),
            pl.BlockSpec((1, 256), lambda n: (0, 0)),
            pl.BlockSpec((1, 256), lambda n: (0, 0)),
        ],
        out_specs=pl.BlockSpec((2, _N_BLK, 128), lambda n: (0, n, 0)),
        out_shape=jax.ShapeDtypeStruct((2, N, 128), jnp.float32),
    )(x, scale.reshape(1, 256), shift.reshape(1, 256))


def _mm_chunks(A, W, init=None, scale=None, shift=None, relu=False):
    K = A.shape[0]
    n_oc, Fo = W.shape[0], W.shape[3]
    Htot = n_oc * Fo
    with_bn = scale is not None
    with_init = init is not None

    def body(*refs):
        refs = list(refs)
        a_ref = refs.pop(0)
        w_ref = refs.pop(0)
        i_ref = refs.pop(0) if with_init else None
        s_ref = refs.pop(0) if with_bn else None
        t_ref = refs.pop(0) if with_bn else None
        o_ref = refs.pop(0)
        acc = lax.dot_general(
            a_ref[0], w_ref[0, 0], (((1,), (0,)), ((), ())),
            precision=lax.Precision.DEFAULT,
            preferred_element_type=jnp.float32)
        for k in range(1, K):
            acc = acc + lax.dot_general(
                a_ref[k], w_ref[0, k], (((1,), (0,)), ((), ())),
                precision=lax.Precision.DEFAULT,
                preferred_element_type=jnp.float32)
        if with_init:
            acc = acc + i_ref[0]
        if with_bn:
            acc = acc * s_ref[...] + t_ref[...]
            if relu:
                acc = jnp.maximum(acc, 0.0)
        o_ref[0] = acc

    in_specs = [
        pl.BlockSpec((K, _N_BLK, 128), lambda n, h: (0, n, 0)),
        pl.BlockSpec((1, K, 128, Fo), lambda n, h: (h, 0, 0, 0)),
    ]
    args = [A, W]
    if with_init:
        in_specs.append(pl.BlockSpec((1, _N_BLK, Fo), lambda n, h: (h, n, 0)))
        args.append(init)
    if with_bn:
        in_specs += [
            pl.BlockSpec((1, Fo), lambda n, h: (0, h)),
            pl.BlockSpec((1, Fo), lambda n, h: (0, h)),
        ]
        args += [scale.reshape(1, Htot), shift.reshape(1, Htot)]
    return pl.pallas_call(
        body,
        grid=(N // _N_BLK, n_oc),
        in_specs=in_specs,
        out_specs=pl.BlockSpec((1, _N_BLK, Fo), lambda n, h: (h, n, 0)),
        out_shape=jax.ShapeDtypeStruct((n_oc, N, Fo), jnp.float32),
    )(*args)


def _layer3(sA, pre2, U2w, Wp, scale, shift):
    def body(a_ref, i_ref, u_ref, w_ref, s_ref, t_ref, o_ref):
        z = jnp.concatenate([i_ref[c] for c in range(4)], axis=1)
        for k in range(8):
            z = z + lax.dot_general(
                a_ref[k], u_ref[k], (((1,), (0,)), ((), ())),
                precision=lax.Precision.DEFAULT,
                preferred_element_type=jnp.float32)
        h2 = jnp.maximum(z * s_ref[...] + t_ref[...], 0.0)
        P = lax.dot_general(
            h2, w_ref[...], (((1,), (0,)), ((), ())),
            precision=lax.Precision.DEFAULT,
            preferred_element_type=jnp.float32)
        o_ref[0] = P[:, :128]
        o_ref[1] = P[:, 128:256]
        o_ref[2] = P[:, 256:]

    return pl.pallas_call(
        body,
        grid=(N // _N_BLK,),
        in_specs=[
            pl.BlockSpec((8, _N_BLK, 128), lambda n: (0, n, 0)),
            pl.BlockSpec((4, _N_BLK, 128), lambda n: (0, n, 0)),
            pl.BlockSpec((8, 128, 512), lambda n: (0, 0, 0)),
            pl.BlockSpec((512, 384), lambda n: (0, 0)),
            pl.BlockSpec((1, 512), lambda n: (0, 0)),
            pl.BlockSpec((1, 512), lambda n: (0, 0)),
        ],
        out_specs=pl.BlockSpec((3, _N_BLK, 128), lambda n: (0, n, 0)),
        out_shape=jax.ShapeDtypeStruct((3, N, 128), jnp.float32),
    )(sA, pre2, U2w, Wp, scale.reshape(1, 512), shift.reshape(1, 512))


def _combine(sf, p2):
    def body(a_ref, b_ref, o_ref):
        o_ref[...] = (a_ref[0] + a_ref[1] + b_ref[...])[:, :64]

    return pl.pallas_call(
        body,
        grid=(N // _N_BLK,),
        in_specs=[
            pl.BlockSpec((2, _N_BLK, 128), lambda n: (0, n, 0)),
            pl.BlockSpec((_N_BLK, 128), lambda n: (n, 0)),
        ],
        out_specs=pl.BlockSpec((_N_BLK, 64), lambda n: (n, 0)),
        out_shape=jax.ShapeDtypeStruct((N, 64), jnp.float32),
    )(sf, p2)


def kernel(x, edge_index0, edge_weight0, edge_index1, edge_weight1,
           W00, W01, W10, W11, W20, W21,
           g1, b1, m1, v1, g2, b2, m2, v2):
    _spmm2, _spmm4, _spmmF = _spmm_kernels()
    c1 = (1.0 - ALPHA) / 2.0
    c2 = ALPHA / 2.0
    s1 = g1 * lax.rsqrt(v1 + EPS)
    t1 = b1 - m1 * s1
    s2 = g2 * lax.rsqrt(v2 + EPS)
    t2 = b2 - m2 * s2
    U1 = jnp.concatenate([c1 * W00, c1 * W01],
                         axis=0).reshape(4, 128, 4, 128).transpose(2, 0, 1, 3)
    V0 = (c2 * (W00 + W01)).reshape(2, 128, 4, 128).transpose(2, 0, 1, 3)
    U2w = jnp.concatenate([c1 * W10, c1 * W11], axis=0).reshape(8, 128, 512)
    V1 = (c2 * (W10 + W11)).reshape(4, 128, 4, 128).transpose(2, 0, 1, 3)
    Wp = jnp.concatenate([
        jnp.pad(c1 * W20, ((0, 0), (0, 64))),
        jnp.pad(c1 * W21, ((0, 0), (0, 64))),
        jnp.pad(c2 * (W20 + W21), ((0, 0), (0, 64))),
    ], axis=1)
    pad_idx = (jnp.arange(_EP - E, dtype=jnp.int32) % N)
    pad_w = jnp.zeros((_EP - E,), jnp.float32)
    srcs = jnp.stack([
        jnp.concatenate([edge_index0[1], pad_idx]),
        jnp.concatenate([edge_index1[1], pad_idx]),
    ]).reshape(2, _NS, _NB, _RG, 80)
    dsts = jnp.stack([
        jnp.concatenate([edge_index0[0], pad_idx]),
        jnp.concatenate([edge_index1[0], pad_idx]),
    ]).reshape(2, _NS, _NB, _RG, 80)
    ws = jnp.stack([
        jnp.concatenate([edge_weight0, pad_w]),
        jnp.concatenate([edge_weight1, pad_w]),
    ]).reshape(2, _NS, _NB, _RG, 80)

    h0 = _chunk_bn(x, s1, t1)
    s0 = _spmm2(h0.reshape(2 * N, 128), srcs, dsts, ws).reshape(4, N, 128)
    pre1 = _mm_chunks(h0, V0)
    h1 = _mm_chunks(s0, U1, init=pre1, scale=s2, shift=t2, relu=True)
    sA = _spmm4(h1.reshape(4 * N, 128), srcs, dsts, ws).reshape(8, N, 128)
    pre2 = _mm_chunks(h1, V1)
    P = _layer3(sA, pre2, U2w, Wp, s2, t2)
    sf = _spmmF(P[:2].reshape(2 * N, 128), srcs, dsts, ws).reshape(2, N, 128)
    return _combine(sf, P[2])


# speedup vs baseline: 1.0750x; 1.0750x over previous
import functools

import jax
import jax.numpy as jnp
from jax import lax
from jax.experimental import pallas as pl
from jax.experimental.pallas import tpu as pltpu
from jax.experimental.pallas import tpu_sc as plsc

N = 10000
E = 160000
ALPHA = 0.1
EPS = 1e-5

_NC = 2
_NS = 16
_EPT = 10240
_EP = _EPT * _NS
_RG = 8
_NB = _EPT // (80 * _RG)
_NR = _EPT // 80
_CP = 624
_RING = 4


def _mesh():
    return plsc.VectorSubcoreMesh(core_axis_name="c", subcore_axis_name="s")


def _sc_scratch(F):
    return [
        pltpu.VMEM_SHARED((N, F), jnp.float32),
        pltpu.VMEM((2, _RG, 80), jnp.int32),
        pltpu.VMEM((2, _RG, 80), jnp.int32),
        pltpu.VMEM((2, _RG, 80), jnp.float32),
        pltpu.VMEM((_RING * 80, F), jnp.float32),
        pltpu.SemaphoreType.DMA((_RING,)),
        pltpu.SemaphoreType.DMA((_RING,)),
        pltpu.SemaphoreType.DMA,
    ]


def _stage_idx(srcs, dsts, ws, src_v, dst_v, w_v, isem, *, g, sid, b, pb,
               tab_off):
    pltpu.async_copy(srcs.at[g, sid, b], src_v.at[pb], isem).wait()
    pltpu.async_copy(dsts.at[g, sid, b], dst_v.at[pb], isem).wait()
    pltpu.async_copy(ws.at[g, sid, b], w_v.at[pb], isem).wait()

    def adj(i, carry):
        for j in range(80 // 16):
            sl = (pb, i, pl.ds(j * 16, 16))
            src_v[sl] = src_v[sl] + tab_off
        return carry

    lax.fori_loop(0, _RG, adj, 0)


def _spmm_pass(table, srcs, dsts, ws, out, acc, src_v, dst_v, w_v, rows_v,
               gsem, ssem, isem, *, g, sid, tab_off, out_off, fv):
    zvec = jnp.zeros((16,), jnp.float32)

    def zrow(i, carry):
        for j in range(fv):
            rows_v[i, pl.ds(j * 16, 16)] = zvec
        return carry

    lax.fori_loop(0, 80, zrow, 0)
    zsrc = rows_v.at[pl.ds(0, 80)]
    for zb in range(7):
        pltpu.async_copy(zsrc, acc.at[pl.ds(sid * _CP + zb * 80, 80)],
                         isem).wait()
    pltpu.async_copy(rows_v.at[pl.ds(0, 64)],
                     acc.at[pl.ds(sid * _CP + 560, 64)], isem).wait()

    @pl.when(sid == _NS - 1)
    def _():
        pltpu.async_copy(zsrc, acc.at[pl.ds(N - 80, 80)], isem).wait()

    plsc.subcore_barrier()

    def gather_desc(t, s):
        b = t // _RG
        pb = lax.rem(b, 2)
        r = t - b * _RG
        return pltpu.make_async_copy(
            table.at[src_v.at[pb, r]],
            rows_v.at[pl.ds(s * 80, 80)], gsem.at[s])

    def scatter_desc(t, s):
        b = t // _RG
        pb = lax.rem(b, 2)
        r = t - b * _RG
        return pltpu.make_async_copy(
            rows_v.at[pl.ds(s * 80, 80)],
            acc.at[dst_v.at[pb, r]], ssem.at[s])

    _stage_idx(srcs, dsts, ws, src_v, dst_v, w_v, isem, g=g, sid=sid,
               b=0, pb=0, tab_off=tab_off)
    gather_desc(0, 0).start()
    gather_desc(1, 1).start()
    gather_desc(2, 2).start()

    def step(t, carry):
        s = lax.rem(t, _RING)
        gather_desc(t, s).wait()

        b = t // _RG
        r = t - b * _RG

        pb = lax.rem(b, 2)

        for q in range(80 // 16):
            wvec = w_v[pb, r, pl.ds(q * 16, 16)]
            for l in range(16):
                wspl = wvec.at[jnp.full((16,), l, jnp.int32)].get(
                    mode="promise_in_bounds")
                i = q * 16 + l
                for j in range(fv):
                    sl2 = (s * 80 + i, pl.ds(j * 16, 16))
                    rows_v[sl2] = rows_v[sl2] * wspl

        @pl.when(t + 3 < _NR)
        def _():
            s2 = lax.rem(t + 3, _RING)

            @pl.when(t >= 1)
            def _():
                scatter_desc(t - 1, s2).wait()

            @pl.when(lax.rem(t + 3, _RG) == 0)
            def _():
                b2 = (t + 3) // _RG
                _stage_idx(srcs, dsts, ws, src_v, dst_v, w_v, isem, g=g,
                           sid=sid, b=b2, pb=lax.rem(b2, 2),
                           tab_off=tab_off)

            gather_desc(t + 3, s2).start()

        scatter_desc(t, s).start(add=True)
        return carry

    lax.fori_loop(0, _NR, step, 0)
    for dt in range(_RING):
        t = _NR - _RING + dt
        scatter_desc(t, t % _RING).wait()
    plsc.subcore_barrier()
    pltpu.async_copy(acc.at[pl.ds(sid * _CP, _CP)],
                     out.at[pl.ds(out_off + sid * _CP, _CP)], isem).wait()

    @pl.when(sid == _NS - 1)
    def _():
        pltpu.async_copy(acc.at[pl.ds(_NS * _CP, N - _NS * _CP)],
                         out.at[pl.ds(out_off + _NS * _CP, N - _NS * _CP)],
                         isem).wait()

    plsc.subcore_barrier()


def _make_spmm2g(n_ch, F):
    n_pass = n_ch // _NC
    fv = F // 16

    @functools.partial(
        pl.kernel, mesh=_mesh(),
        out_type=jax.ShapeDtypeStruct((2 * n_ch * N, F), jnp.float32),
        scratch_types=_sc_scratch(F),
    )
    def spmm(table, srcs, dsts, ws, out, acc, src_v, dst_v, w_v, rows_v,
             gsem, ssem, isem):
        cid = lax.axis_index("c")
        sid = lax.axis_index("s")

        def one_pass(q, carry):
            g = q // n_pass
            chunk = (q % n_pass) * _NC + cid
            _spmm_pass(table, srcs, dsts, ws, out, acc, src_v, dst_v,
                       w_v, rows_v, gsem, ssem, isem, g=g, sid=sid,
                       tab_off=chunk * N,
                       out_off=(g * n_ch + chunk) * N, fv=fv)
            return carry

        lax.fori_loop(0, 2 * n_pass, one_pass, 0)

    return spmm


def _make_spmm_final(F):
    fv = F // 16

    @functools.partial(
        pl.kernel, mesh=_mesh(),
        out_type=jax.ShapeDtypeStruct((2 * N, F), jnp.float32),
        scratch_types=_sc_scratch(F),
    )
    def spmmf(table, srcs, dsts, ws, out, acc, src_v, dst_v, w_v, rows_v,
              gsem, ssem, isem):
        cid = lax.axis_index("c")
        sid = lax.axis_index("s")
        _spmm_pass(table, srcs, dsts, ws, out, acc, src_v, dst_v, w_v,
                   rows_v, gsem, ssem, isem, g=cid, sid=sid,
                   tab_off=cid * N, out_off=cid * N, fv=fv)

    return spmmf


@functools.lru_cache(maxsize=1)
def _spmm_kernels():
    return _make_spmm2g(2, 128), _make_spmm2g(4, 128), _make_spmm_final(128)


_N_BLK = 400


def _chunk_bn(x, scale, shift):
    def body(x_ref, s_ref, t_ref, o_ref):
        y = x_ref[...] * s_ref[...] + t_ref[...]
        o_ref[0] = y[:, :128]
        o_ref[1] = y[:, 128:]

    return pl.pallas_call(
        body,
        grid=(N // _N_BLK,),
        in_specs=[
            pl.BlockSpec((_N_BLK, 256), lambda n: (n, 0)),
            pl.BlockSpec((1, 256), lambda n: (0, 0)),
            pl.BlockSpec((1, 256), lambda n: (0, 0)),
        ],
        out_specs=pl.BlockSpec((2, _N_BLK, 128), lambda n: (0, n, 0)),
        out_shape=jax.ShapeDtypeStruct((2, N, 128), jnp.float32),
    )(x, scale.reshape(1, 256), shift.reshape(1, 256))


def _mm_chunks(A, W, init=None, scale=None, shift=None, relu=False):
    K = A.shape[0]
    n_oc, Fo = W.shape[0], W.shape[3]
    Htot = n_oc * Fo
    with_bn = scale is not None
    with_init = init is not None

    def body(*refs):
        refs = list(refs)
        a_ref = refs.pop(0)
        w_ref = refs.pop(0)
        i_ref = refs.pop(0) if with_init else None
        s_ref = refs.pop(0) if with_bn else None
        t_ref = refs.pop(0) if with_bn else None
        o_ref = refs.pop(0)
        acc = lax.dot_general(
            a_ref[0], w_ref[0, 0], (((1,), (0,)), ((), ())),
            precision=lax.Precision.DEFAULT,
            preferred_element_type=jnp.float32)
        for k in range(1, K):
            acc = acc + lax.dot_general(
                a_ref[k], w_ref[0, k], (((1,), (0,)), ((), ())),
                precision=lax.Precision.DEFAULT,
                preferred_element_type=jnp.float32)
        if with_init:
            acc = acc + i_ref[0]
        if with_bn:
            acc = acc * s_ref[...] + t_ref[...]
            if relu:
                acc = jnp.maximum(acc, 0.0)
        o_ref[0] = acc

    in_specs = [
        pl.BlockSpec((K, _N_BLK, 128), lambda n, h: (0, n, 0)),
        pl.BlockSpec((1, K, 128, Fo), lambda n, h: (h, 0, 0, 0)),
    ]
    args = [A, W]
    if with_init:
        in_specs.append(pl.BlockSpec((1, _N_BLK, Fo), lambda n, h: (h, n, 0)))
        args.append(init)
    if with_bn:
        in_specs += [
            pl.BlockSpec((1, Fo), lambda n, h: (0, h)),
            pl.BlockSpec((1, Fo), lambda n, h: (0, h)),
        ]
        args += [scale.reshape(1, Htot), shift.reshape(1, Htot)]
    return pl.pallas_call(
        body,
        grid=(N // _N_BLK, n_oc),
        in_specs=in_specs,
        out_specs=pl.BlockSpec((1, _N_BLK, Fo), lambda n, h: (h, n, 0)),
        out_shape=jax.ShapeDtypeStruct((n_oc, N, Fo), jnp.float32),
    )(*args)


def _layer3(sA, pre2, U2w, Wp, scale, shift):
    def body(a_ref, i_ref, u_ref, w_ref, s_ref, t_ref, o_ref):
        z = jnp.concatenate([i_ref[c] for c in range(4)], axis=1)
        for k in range(8):
            z = z + lax.dot_general(
                a_ref[k], u_ref[k], (((1,), (0,)), ((), ())),
                precision=lax.Precision.DEFAULT,
                preferred_element_type=jnp.float32)
        h2 = jnp.maximum(z * s_ref[...] + t_ref[...], 0.0)
        P = lax.dot_general(
            h2, w_ref[...], (((1,), (0,)), ((), ())),
            precision=lax.Precision.DEFAULT,
            preferred_element_type=jnp.float32)
        o_ref[0] = P[:, :128]
        o_ref[1] = P[:, 128:256]
        o_ref[2] = P[:, 256:]

    return pl.pallas_call(
        body,
        grid=(N // _N_BLK,),
        in_specs=[
            pl.BlockSpec((8, _N_BLK, 128), lambda n: (0, n, 0)),
            pl.BlockSpec((4, _N_BLK, 128), lambda n: (0, n, 0)),
            pl.BlockSpec((8, 128, 512), lambda n: (0, 0, 0)),
            pl.BlockSpec((512, 384), lambda n: (0, 0)),
            pl.BlockSpec((1, 512), lambda n: (0, 0)),
            pl.BlockSpec((1, 512), lambda n: (0, 0)),
        ],
        out_specs=pl.BlockSpec((3, _N_BLK, 128), lambda n: (0, n, 0)),
        out_shape=jax.ShapeDtypeStruct((3, N, 128), jnp.float32),
    )(sA, pre2, U2w, Wp, scale.reshape(1, 512), shift.reshape(1, 512))


def _combine(sf, p2):
    def body(a_ref, b_ref, o_ref):
        o_ref[...] = (a_ref[0] + a_ref[1] + b_ref[...])[:, :64]

    return pl.pallas_call(
        body,
        grid=(N // _N_BLK,),
        in_specs=[
            pl.BlockSpec((2, _N_BLK, 128), lambda n: (0, n, 0)),
            pl.BlockSpec((_N_BLK, 128), lambda n: (n, 0)),
        ],
        out_specs=pl.BlockSpec((_N_BLK, 64), lambda n: (n, 0)),
        out_shape=jax.ShapeDtypeStruct((N, 64), jnp.float32),
    )(sf, p2)


def kernel(x, edge_index0, edge_weight0, edge_index1, edge_weight1,
           W00, W01, W10, W11, W20, W21,
           g1, b1, m1, v1, g2, b2, m2, v2):
    _spmm2, _spmm4, _spmmF = _spmm_kernels()
    c1 = (1.0 - ALPHA) / 2.0
    c2 = ALPHA / 2.0
    s1 = g1 * lax.rsqrt(v1 + EPS)
    t1 = b1 - m1 * s1
    s2 = g2 * lax.rsqrt(v2 + EPS)
    t2 = b2 - m2 * s2
    U1 = jnp.concatenate([c1 * W00, c1 * W01],
                         axis=0).reshape(4, 128, 4, 128).transpose(2, 0, 1, 3)
    V0 = (c2 * (W00 + W01)).reshape(2, 128, 4, 128).transpose(2, 0, 1, 3)
    U2w = jnp.concatenate([c1 * W10, c1 * W11], axis=0).reshape(8, 128, 512)
    V1 = (c2 * (W10 + W11)).reshape(4, 128, 4, 128).transpose(2, 0, 1, 3)
    Wp = jnp.concatenate([
        jnp.pad(c1 * W20, ((0, 0), (0, 64))),
        jnp.pad(c1 * W21, ((0, 0), (0, 64))),
        jnp.pad(c2 * (W20 + W21), ((0, 0), (0, 64))),
    ], axis=1)
    pad_idx = (jnp.arange(_EP - E, dtype=jnp.int32) % N)
    pad_w = jnp.zeros((_EP - E,), jnp.float32)
    srcs = jnp.stack([
        jnp.concatenate([edge_index0[1], pad_idx]),
        jnp.concatenate([edge_index1[1], pad_idx]),
    ]).reshape(2, _NS, _NB, _RG, 80)
    dsts = jnp.stack([
        jnp.concatenate([edge_index0[0], pad_idx]),
        jnp.concatenate([edge_index1[0], pad_idx]),
    ]).reshape(2, _NS, _NB, _RG, 80)
    ws = jnp.stack([
        jnp.concatenate([edge_weight0, pad_w]),
        jnp.concatenate([edge_weight1, pad_w]),
    ]).reshape(2, _NS, _NB, _RG, 80)

    h0 = _chunk_bn(x, s1, t1)
    s0 = _spmm2(h0.reshape(2 * N, 128), srcs, dsts, ws).reshape(4, N, 128)
    pre1 = _mm_chunks(h0, V0)
    h1 = _mm_chunks(s0, U1, init=pre1, scale=s2, shift=t2, relu=True)
    sA = _spmm4(h1.reshape(4 * N, 128), srcs, dsts, ws).reshape(8, N, 128)
    pre2 = _mm_chunks(h1, V1)
    P = _layer3(sA, pre2, U2w, Wp, s2, t2)
    sf = _spmmF(P[:2].reshape(2 * N, 128), srcs, dsts, ws).reshape(2, N, 128)
    return _combine(sf, P[2])
